# baseline (speedup 1.0000x reference)
import functools

import jax
import jax.numpy as jnp
from jax import lax
from jax.experimental import pallas as pl
from jax.experimental.pallas import tpu as pltpu
from jax.experimental.pallas import tpu_sc as plsc

B = 4096
T = 200
D = 64
NW = 32
L = 16
TJ = T // 8
BJ = B // 128
NCHUNK = T
OSTRIDE = 129


def _body(p4_hbm, q4_hbm, post_hbm, table2_hbm, out_hbm,
          pidx_v, qoff_v, post_v, rows0, rows1, obuf0, obuf1,
          sg0, sg1, so0, so1):
    w = lax.axis_index("s") * 2 + lax.axis_index("c")
    rows = (rows0, rows1)
    obuf = (obuf0, obuf1)
    sg = (sg0, sg1)
    so = (so0, so1)

    pltpu.sync_copy(post_hbm, post_v)
    pltpu.sync_copy(p4_hbm.at[:, w], pidx_v)
    pltpu.sync_copy(q4_hbm.at[:, w], qoff_v)

    iota = lax.iota(jnp.int32, L)

    def tj_tt(c):
        return lax.shift_right_logical(c, 3), lax.rem(c, 8)

    def issue_gather(c, k):
        tj, tt = tj_tt(c)
        pltpu.async_copy(table2_hbm.at[pidx_v.at[tj, tt]], rows[k], sg[k])

    def wait_gather(k):
        pltpu.make_async_copy(table2_hbm.at[pidx_v.at[0, 0]],
                              rows[k], sg[k]).wait()

    def issue_out(c, k):
        tj, tt = tj_tt(c)
        pltpu.async_copy(obuf[k].at[:, pl.ds(0, 128)],
                         out_hbm.at[:, tj, w, pl.ds(tt * 128, 128)], so[k])

    def wait_out(k):
        pltpu.make_async_copy(obuf[k].at[:, pl.ds(0, 128)],
                              out_hbm.at[:, 0, 0, pl.ds(0, 128)], so[k]).wait()

    issue_gather(0, 0)
    issue_gather(1, 1)

    def loop_body(j, carry):
        for k in (0, 1):
            c = 2 * j + k
            tj, tt = tj_tt(c)
            t = lax.add(lax.mul(tj, 8), tt)
            wait_gather(k)

            @pl.when(c >= 2)
            def _():
                wait_out(k)

            pvecs = [post_v[t, pl.ds(dc * L, L)] for dc in range(D // L)]

            @plsc.parallel_loop(0, 128, L, unroll=2)
            def per_block(b0):
                qv = qoff_v[tj, tt, pl.ds(b0, L)]
                for l in range(L):
                    q = qv[l]
                    bb = lax.add(b0, l)
                    bvec = jnp.zeros((L,), jnp.int32) + bb
                    for dc in range(D // L):
                        v = rows[k][bb, pl.ds(q + dc * L, L)]
                        plsc.store_scatter(obuf[k], [dc * L + iota, bvec],
                                           v + pvecs[dc])

            issue_out(c, k)

            @pl.when(c + 2 < NCHUNK)
            def _():
                issue_gather(c + 2, k)
        return carry

    lax.fori_loop(0, NCHUNK // 2, loop_body, 0, unroll=False)
    wait_out(0)
    wait_out(1)


_RGRID = 489
_RROWS = _RGRID * 1024


def _tc_body(in_ref, o_ref):
    x = in_ref[...]
    eye = jnp.eye(D, dtype=jnp.float32)
    dn = (((0,), (0,)), ((), ()))
    o_ref[:, 0:D] = lax.dot_general(
        x[:, 0:1024], eye, dn, precision=lax.Precision.HIGHEST,
        preferred_element_type=jnp.float32)
    o_ref[:, D:128] = lax.dot_general(
        x[:, 1024:2048], eye, dn, precision=lax.Precision.HIGHEST,
        preferred_element_type=jnp.float32)


def _repack_table(table):
    tt = jnp.transpose(table.astype(jnp.float32), (1, 0))
    return pl.pallas_call(
        _tc_body,
        grid=(_RGRID,),
        in_specs=[pl.BlockSpec((D, 2048), lambda i: (0, i))],
        out_specs=pl.BlockSpec((1024, 128), lambda i: (i, 0)),
        out_shape=jax.ShapeDtypeStruct((_RROWS, 128), jnp.float32),
    )(tt)


def kernel(rid, table, pos):
    rid4 = (rid.astype(jnp.int32)
            .reshape(BJ, 128, TJ, 8).transpose(2, 0, 3, 1))
    p4 = ((rid4 >> 11) << 10) | (rid4 & 1023)
    q4 = ((rid4 >> 10) & 1) * D
    post = jnp.transpose(pos.astype(jnp.float32).reshape(D, T), (1, 0))

    mesh = plsc.VectorSubcoreMesh(core_axis_name="c", subcore_axis_name="s")
    run = functools.partial(
        pl.kernel,
        mesh=mesh,
        out_type=jax.ShapeDtypeStruct((D, TJ, BJ, 1024), jnp.float32),
        scratch_types=[
            pltpu.VMEM((TJ, 8, 128), jnp.int32),
            pltpu.VMEM((TJ, 8, 128), jnp.int32),
            pltpu.VMEM((T, D), jnp.float32),
            pltpu.VMEM((128, 128), jnp.float32),
            pltpu.VMEM((128, 128), jnp.float32),
            pltpu.VMEM((D, OSTRIDE), jnp.float32),
            pltpu.VMEM((D, OSTRIDE), jnp.float32),
            pltpu.SemaphoreType.DMA,
            pltpu.SemaphoreType.DMA,
            pltpu.SemaphoreType.DMA,
            pltpu.SemaphoreType.DMA,
        ],
        compiler_params=pltpu.CompilerParams(needs_layout_passes=False,
                                             use_tc_tiling_on_sc=False),
    )(_body)
    out5 = run(p4, q4, post, _repack_table(table))
    return (out5.reshape(D, TJ, BJ, 8, 128)
            .transpose(2, 4, 0, 1, 3)
            .reshape(B, D, T))


# speedup vs baseline: 1.4000x; 1.4000x over previous
import functools

import jax
import jax.numpy as jnp
from jax import lax
from jax.experimental import pallas as pl
from jax.experimental.pallas import tpu as pltpu
from jax.experimental.pallas import tpu_sc as plsc

B = 4096
T = 200
D = 64
NW = 32
L = 16
TJ = T // 8
BJ = B // 128
NCHUNK = T // 2
OSTRIDE = 257


def _body(rid_hbm, post_hbm, table_hbm, out_hbm,
          idx_v, post_v, rows0, rows1, obuf0, obuf1,
          sg0, sg1, so0, so1):
    w = lax.axis_index("s") * 2 + lax.axis_index("c")
    rows = (rows0, rows1)
    obuf = (obuf0, obuf1)
    sg = (sg0, sg1)
    so = (so0, so1)

    pltpu.sync_copy(post_hbm, post_v)
    pltpu.sync_copy(rid_hbm.at[:, w], idx_v)

    iota = lax.iota(jnp.int32, L)

    def tj_tt(c):
        return lax.shift_right_logical(c, 2), lax.mul(lax.rem(c, 4), 2)

    def issue_gather(c, k):
        tj, tt0 = tj_tt(c)
        for h in (0, 1):
            pltpu.async_copy(table_hbm.at[idx_v.at[tj, tt0 + h]],
                             rows[k].at[h], sg[k])

    def wait_gather(k):
        for h in (0, 1):
            pltpu.make_async_copy(table_hbm.at[idx_v.at[0, 0]],
                                  rows[k].at[h], sg[k]).wait()

    def issue_out(c, k):
        tj, tt0 = tj_tt(c)
        pltpu.async_copy(obuf[k].at[:, pl.ds(0, 256)],
                         out_hbm.at[:, tj, w, pl.ds(tt0 * 128, 256)], so[k])

    def wait_out(k):
        pltpu.make_async_copy(obuf[k].at[:, pl.ds(0, 256)],
                              out_hbm.at[:, 0, 0, pl.ds(0, 256)], so[k]).wait()

    issue_gather(0, 0)
    issue_gather(1, 1)

    def loop_body(j, carry):
        for k in (0, 1):
            c = 2 * j + k
            tj, tt0 = tj_tt(c)
            wait_gather(k)

            @pl.when(c >= 2)
            def _():
                wait_out(k)

            for h in (0, 1):
                t = lax.add(lax.add(lax.mul(tj, 8), tt0), h)
                pvecs = [post_v[t, pl.ds(dc * L, L)] for dc in range(D // L)]
                col0 = h * 128

                @plsc.parallel_loop(0, 128, 1, unroll=8)
                def per_bb(bb):
                    bvec = jnp.zeros((L,), jnp.int32) + (col0 + bb)
                    for dc in range(D // L):
                        v = rows[k][h, bb, pl.ds(dc * L, L)]
                        plsc.store_scatter(obuf[k], [dc * L + iota, bvec],
                                           v + pvecs[dc])

            issue_out(c, k)

            @pl.when(c + 2 < NCHUNK)
            def _():
                issue_gather(c + 2, k)
        return carry

    lax.fori_loop(0, NCHUNK // 2, loop_body, 0, unroll=False)
    wait_out(0)
    wait_out(1)


def kernel(rid, table, pos):
    rid4 = (rid.astype(jnp.int32)
            .reshape(BJ, 128, TJ, 8).transpose(2, 0, 3, 1))
    post = jnp.transpose(pos.astype(jnp.float32).reshape(D, T), (1, 0))

    mesh = plsc.VectorSubcoreMesh(core_axis_name="c", subcore_axis_name="s")
    run = functools.partial(
        pl.kernel,
        mesh=mesh,
        out_type=jax.ShapeDtypeStruct((D, TJ, BJ, 1024), jnp.float32),
        scratch_types=[
            pltpu.VMEM((TJ, 8, 128), jnp.int32),
            pltpu.VMEM((T, D), jnp.float32),
            pltpu.VMEM((2, 128, D), jnp.float32),
            pltpu.VMEM((2, 128, D), jnp.float32),
            pltpu.VMEM((D, OSTRIDE), jnp.float32),
            pltpu.VMEM((D, OSTRIDE), jnp.float32),
            pltpu.SemaphoreType.DMA,
            pltpu.SemaphoreType.DMA,
            pltpu.SemaphoreType.DMA,
            pltpu.SemaphoreType.DMA,
        ],
        compiler_params=pltpu.CompilerParams(needs_layout_passes=False,
                                             use_tc_tiling_on_sc=False),
    )(_body)
    out5 = run(rid4, post, table)
    return (out5.reshape(D, TJ, BJ, 8, 128)
            .transpose(2, 4, 0, 1, 3)
            .reshape(B, D, T))
